# baseline (speedup 1.0000x reference)
import functools

import jax
import jax.numpy as jnp
from jax import lax
from jax.experimental import pallas as pl
from jax.experimental.pallas import tpu as pltpu
from jax.experimental.pallas import tpu_sc as plsc

_BN = 16 * 576
_K = 1024
_D = 256
_BLK = 512
_CC = 128
_TC = 128


def _rowsq(m):
    s = m * m
    h = s[:, :128] + s[:, 128:]
    acc = h[:, 0:8]
    for v in range(1, 16):
        acc = acc + h[:, 8 * v:8 * v + 8]
    t = acc[:, :4] + acc[:, 4:]
    t = t[:, :2] + t[:, 2:]
    return t[:, 0] + t[:, 1]


def _colsq(mT):
    s = mT * mT
    h = s[:128, :] + s[128:, :]
    acc = h[0:8]
    for v in range(1, 16):
        acc = acc + h[8 * v:8 * v + 8]
    t = acc[:4] + acc[4:]
    t = t[:2] + t[2:]
    return t[0:1] + t[1:2]


def _esq_block(e_ref, esq_ref):
    esq_ref[...] = _rowsq(e_ref[...])[:, None]


def _esq(e):
    return pl.pallas_call(
        _esq_block,
        out_shape=jax.ShapeDtypeStruct((_K, 1), jnp.float32),
    )(e)


def _vq_block(x_ref, e_ref, esq_ref, idx_ref):
    xb = x_ref[...]
    esq = esq_ref[...]
    iota0 = jax.lax.broadcasted_iota(jnp.int32, (_CC, _TC), 0)
    e = e_ref[...]

    half = _BLK // 2
    for h in range(2):
        hs = slice(h * half, (h + 1) * half)
        xh = xb[hs, :]
        cross2 = jax.lax.dot_general(
            e, xh + xh, (((1,), (1,)), ((), ())),
            preferred_element_type=jnp.float32)
        xsq = _colsq(xh.T)

        for t in range(half // _TC):
            ts = slice(t * _TC, (t + 1) * _TC)
            xsq_t = xsq[:, ts]
            carry_v = None
            carry_i = None
            for c in range(_K // _CC):
                tile = cross2[c * _CC:(c + 1) * _CC, ts]
                d2 = jnp.maximum(
                    xsq_t - tile + esq[c * _CC:(c + 1) * _CC, :], 0.0)
                v = jnp.sqrt(d2)
                mv = jnp.min(v, axis=0, keepdims=True)
                cand = jnp.where(v == mv, iota0, _K)
                ci = jnp.min(cand, axis=0, keepdims=True) + c * _CC
                if carry_v is None:
                    carry_v, carry_i = mv, ci
                else:
                    better = mv < carry_v
                    carry_i = jnp.where(better, ci, carry_i)
                    carry_v = jnp.minimum(carry_v, mv)
            idx_ref[:, slice(h * half + t * _TC,
                             h * half + (t + 1) * _TC)] = carry_i


def _tc_indices(flat, e, esq):
    idx = pl.pallas_call(
        _vq_block,
        grid=(_BN // _BLK,),
        in_specs=[
            pl.BlockSpec((_BLK, _D), lambda i: (i, 0)),
            pl.BlockSpec((_K, _D), lambda i: (0, 0)),
            pl.BlockSpec((_K, 1), lambda i: (0, 0)),
        ],
        out_specs=pl.BlockSpec((1, _BLK), lambda i: (0, i)),
        out_shape=jax.ShapeDtypeStruct((1, _BN), jnp.int32),
    )(flat, e, esq)
    return idx.reshape(_BN)


_SC_INFO = plsc.get_sparse_core_info()
_NW = _SC_INFO.num_cores * _SC_INFO.num_subcores
_BPW = _BN // _NW


@functools.partial(
    pl.kernel,
    mesh=plsc.VectorSubcoreMesh(core_axis_name="c", subcore_axis_name="s"),
    out_type=jax.ShapeDtypeStruct((_BN, _D), jnp.float32),
    scratch_types=[
        pltpu.VMEM((_BPW,), jnp.int32),
        pltpu.VMEM((_BPW, _D), jnp.float32),
        pltpu.SemaphoreType.DMA,
    ],
)
def _sc_gather(table_hbm, idx_hbm, out_hbm, idx_v, rows_v, sem):
    wid = lax.axis_index("s") * _SC_INFO.num_cores + lax.axis_index("c")
    base = wid * _BPW
    pltpu.sync_copy(idx_hbm.at[pl.ds(base, _BPW)], idx_v)
    pltpu.async_copy(table_hbm.at[idx_v], rows_v, sem).wait()
    pltpu.sync_copy(rows_v, out_hbm.at[pl.ds(base, _BPW)])


@jax.jit
def kernel(x, embed):
    x = x.astype(jnp.float32)
    b, n, d = x.shape
    flat = x.reshape(b * n, d)
    e = embed[0]
    idx = _tc_indices(flat, e, _esq(e))
    quantize = _sc_gather(e, idx)
    return quantize.reshape(b, n, d), idx.reshape(b, n)


# speedup vs baseline: 1.0379x; 1.0379x over previous
import functools

import jax
import jax.numpy as jnp
from jax import lax
from jax.experimental import pallas as pl
from jax.experimental.pallas import tpu as pltpu
from jax.experimental.pallas import tpu_sc as plsc

_BN = 16 * 576
_K = 1024
_D = 256
_BLK = 512
_CC = 128
_TC = 128


def _rowsq(m):
    s = m * m
    h = s[:, :128] + s[:, 128:]
    acc = h[:, 0:8]
    for v in range(1, 16):
        acc = acc + h[:, 8 * v:8 * v + 8]
    t = acc[:, :4] + acc[:, 4:]
    t = t[:, :2] + t[:, 2:]
    return t[:, 0] + t[:, 1]


def _colsq(mT):
    s = mT * mT
    h = s[:128, :] + s[128:, :]
    acc = h[0:8]
    for v in range(1, 16):
        acc = acc + h[8 * v:8 * v + 8]
    t = acc[:4] + acc[4:]
    t = t[:2] + t[2:]
    return t[0:1] + t[1:2]


def _vq_block(x_ref, e_ref, idx_ref, esq_ref):
    @pl.when(pl.program_id(0) == 0)
    def _():
        esq_ref[...] = _rowsq(e_ref[...])[:, None]

    xb = x_ref[...]
    esq = esq_ref[...]
    iota0 = jax.lax.broadcasted_iota(jnp.int32, (_CC, _TC), 0)
    e = e_ref[...]

    half = _BLK // 2
    for h in range(2):
        hs = slice(h * half, (h + 1) * half)
        xh = xb[hs, :]
        cross2 = jax.lax.dot_general(
            e, xh + xh, (((1,), (1,)), ((), ())),
            preferred_element_type=jnp.float32)
        xsq = _colsq(xh.T)

        for t in range(half // _TC):
            ts = slice(t * _TC, (t + 1) * _TC)
            xsq_t = xsq[:, ts]
            carry_v = None
            carry_i = None
            for c in range(_K // _CC):
                tile = cross2[c * _CC:(c + 1) * _CC, ts]
                d2 = jnp.maximum(
                    xsq_t - tile + esq[c * _CC:(c + 1) * _CC, :], 0.0)
                v = jnp.sqrt(d2)
                mv = jnp.min(v, axis=0, keepdims=True)
                cand = jnp.where(v == mv, iota0, _K)
                ci = jnp.min(cand, axis=0, keepdims=True) + c * _CC
                if carry_v is None:
                    carry_v, carry_i = mv, ci
                else:
                    better = mv < carry_v
                    carry_i = jnp.where(better, ci, carry_i)
                    carry_v = jnp.minimum(carry_v, mv)
            idx_ref[:, slice(h * half + t * _TC,
                             h * half + (t + 1) * _TC)] = carry_i


def _tc_indices(flat, e):
    idx = pl.pallas_call(
        _vq_block,
        grid=(_BN // _BLK,),
        in_specs=[
            pl.BlockSpec((_BLK, _D), lambda i: (i, 0)),
            pl.BlockSpec((_K, _D), lambda i: (0, 0)),
        ],
        out_specs=pl.BlockSpec((1, _BLK), lambda i: (0, i)),
        out_shape=jax.ShapeDtypeStruct((1, _BN), jnp.int32),
        scratch_shapes=[pltpu.VMEM((_K, 1), jnp.float32)],
    )(flat, e)
    return idx.reshape(_BN)


_SC_INFO = plsc.get_sparse_core_info()
_NW = _SC_INFO.num_cores * _SC_INFO.num_subcores
_BPW = _BN // _NW


@functools.partial(
    pl.kernel,
    mesh=plsc.VectorSubcoreMesh(core_axis_name="c", subcore_axis_name="s"),
    out_type=jax.ShapeDtypeStruct((_BN, _D), jnp.float32),
    scratch_types=[
        pltpu.VMEM((_BPW,), jnp.int32),
        pltpu.VMEM((_BPW, _D), jnp.float32),
        pltpu.SemaphoreType.DMA,
    ],
)
def _sc_gather(table_hbm, idx_hbm, out_hbm, idx_v, rows_v, sem):
    wid = lax.axis_index("s") * _SC_INFO.num_cores + lax.axis_index("c")
    base = wid * _BPW
    pltpu.sync_copy(idx_hbm.at[pl.ds(base, _BPW)], idx_v)
    pltpu.async_copy(table_hbm.at[idx_v], rows_v, sem).wait()
    pltpu.sync_copy(rows_v, out_hbm.at[pl.ds(base, _BPW)])


@jax.jit
def kernel(x, embed):
    x = x.astype(jnp.float32)
    b, n, d = x.shape
    flat = x.reshape(b * n, d)
    e = embed[0]
    idx = _tc_indices(flat, e)
    quantize = _sc_gather(e, idx)
    return quantize.reshape(b, n, d), idx.reshape(b, n)


# speedup vs baseline: 1.0398x; 1.0018x over previous
import functools

import jax
import jax.numpy as jnp
from jax import lax
from jax.experimental import pallas as pl
from jax.experimental.pallas import tpu as pltpu
from jax.experimental.pallas import tpu_sc as plsc

_BN = 16 * 576
_K = 1024
_D = 256
_BLK = 512
_CC = 128
_TC = 256


def _rowsq(m):
    s = m * m
    h = s[:, :128] + s[:, 128:]
    acc = h[:, 0:8]
    for v in range(1, 16):
        acc = acc + h[:, 8 * v:8 * v + 8]
    t = acc[:, :4] + acc[:, 4:]
    t = t[:, :2] + t[:, 2:]
    return t[:, 0] + t[:, 1]


def _colsq(mT):
    s = mT * mT
    h = s[:128, :] + s[128:, :]
    acc = h[0:8]
    for v in range(1, 16):
        acc = acc + h[8 * v:8 * v + 8]
    t = acc[:4] + acc[4:]
    t = t[:2] + t[2:]
    return t[0:1] + t[1:2]


def _vq_block(x_ref, e_ref, idx_ref, esq_ref):
    @pl.when(pl.program_id(0) == 0)
    def _():
        esq_ref[...] = _rowsq(e_ref[...])[:, None]

    xb = x_ref[...]
    esq = esq_ref[...]
    iota0 = jax.lax.broadcasted_iota(jnp.int32, (_CC, _TC), 0)
    e = e_ref[...]

    half = _BLK // 2
    for h in range(2):
        hs = slice(h * half, (h + 1) * half)
        xh = xb[hs, :]
        cross2 = jax.lax.dot_general(
            e, xh + xh, (((1,), (1,)), ((), ())),
            preferred_element_type=jnp.float32)
        xsq = _colsq(xh.T)

        for t in range(half // _TC):
            ts = slice(t * _TC, (t + 1) * _TC)
            xsq_t = xsq[:, ts]
            carry_v = None
            carry_i = None
            for c in range(_K // _CC):
                tile = cross2[c * _CC:(c + 1) * _CC, ts]
                d2 = jnp.maximum(
                    xsq_t - tile + esq[c * _CC:(c + 1) * _CC, :], 0.0)
                v = jnp.sqrt(d2)
                mv = jnp.min(v, axis=0, keepdims=True)
                cand = jnp.where(v == mv, iota0, _K)
                ci = jnp.min(cand, axis=0, keepdims=True) + c * _CC
                if carry_v is None:
                    carry_v, carry_i = mv, ci
                else:
                    better = mv < carry_v
                    carry_i = jnp.where(better, ci, carry_i)
                    carry_v = jnp.minimum(carry_v, mv)
            idx_ref[:, slice(h * half + t * _TC,
                             h * half + (t + 1) * _TC)] = carry_i


def _tc_indices(flat, e):
    idx = pl.pallas_call(
        _vq_block,
        grid=(_BN // _BLK,),
        in_specs=[
            pl.BlockSpec((_BLK, _D), lambda i: (i, 0)),
            pl.BlockSpec((_K, _D), lambda i: (0, 0)),
        ],
        out_specs=pl.BlockSpec((1, _BLK), lambda i: (0, i)),
        out_shape=jax.ShapeDtypeStruct((1, _BN), jnp.int32),
        scratch_shapes=[pltpu.VMEM((_K, 1), jnp.float32)],
    )(flat, e)
    return idx.reshape(_BN)


_SC_INFO = plsc.get_sparse_core_info()
_NW = _SC_INFO.num_cores * _SC_INFO.num_subcores
_BPW = _BN // _NW


@functools.partial(
    pl.kernel,
    mesh=plsc.VectorSubcoreMesh(core_axis_name="c", subcore_axis_name="s"),
    out_type=jax.ShapeDtypeStruct((_BN, _D), jnp.float32),
    scratch_types=[
        pltpu.VMEM((_BPW,), jnp.int32),
        pltpu.VMEM((_BPW, _D), jnp.float32),
        pltpu.SemaphoreType.DMA,
    ],
)
def _sc_gather(table_hbm, idx_hbm, out_hbm, idx_v, rows_v, sem):
    wid = lax.axis_index("s") * _SC_INFO.num_cores + lax.axis_index("c")
    base = wid * _BPW
    pltpu.sync_copy(idx_hbm.at[pl.ds(base, _BPW)], idx_v)
    pltpu.async_copy(table_hbm.at[idx_v], rows_v, sem).wait()
    pltpu.sync_copy(rows_v, out_hbm.at[pl.ds(base, _BPW)])


@jax.jit
def kernel(x, embed):
    x = x.astype(jnp.float32)
    b, n, d = x.shape
    flat = x.reshape(b * n, d)
    e = embed[0]
    idx = _tc_indices(flat, e)
    quantize = _sc_gather(e, idx)
    return quantize.reshape(b, n, d), idx.reshape(b, n)
